# baseline (speedup 1.0000x reference)
import functools

import jax
import jax.numpy as jnp
from jax import lax
from jax.experimental import pallas as pl
from jax.experimental.pallas import tpu as pltpu
from jax.experimental.pallas import tpu_sc as plsc

N = 50000
D = 128
E = 200000
R = 3
G = 8
DG = D // G

NS = 16
NROW = 98
EPT = NROW * 128
EPAD = NS * EPT

PN = 50432
PADIDX = 50200
SPAN = PN // NS

NPAD_DEG = 51200
SPAN_DEG = NPAD_DEG // NS


def _mesh():
    return plsc.VectorSubcoreMesh(core_axis_name="c", subcore_axis_name="s")


def _deg_body(edges_hbm, deg_hbm, idx2d, ones_v, zbuf, dac0, dac1, dac2):
    side = lax.axis_index("c")
    tid = lax.axis_index("s")
    dacs = [dac0, dac1, dac2]

    def _fill_z(q, carry):
        zbuf[pl.ds(q * 16, 16)] = jnp.zeros((16,), jnp.float32)
        return carry
    lax.fori_loop(0, SPAN_DEG // 16, _fill_z, 0)

    def _fill_o(q, carry):
        ones_v[pl.ds(q * 16, 16)] = jnp.ones((16,), jnp.float32)
        return carry
    lax.fori_loop(0, 8, _fill_o, 0)

    for a in range(R):
        pltpu.sync_copy(zbuf, dacs[a].at[pl.ds(tid * SPAN_DEG, SPAN_DEG)])
    plsc.subcore_barrier()

    for a in range(R):
        pltpu.sync_copy(edges_hbm.at[a].at[side].at[tid], idx2d)

        def _scat(j, carry):
            pltpu.sync_copy(ones_v, dacs[a].at[idx2d.at[j]], add=True)
            return carry
        lax.fori_loop(0, NROW, _scat, 0)
    plsc.subcore_barrier()

    for a in range(R):
        base = (side * R + a) * NPAD_DEG + tid * SPAN_DEG
        pltpu.sync_copy(dacs[a].at[pl.ds(tid * SPAN_DEG, SPAN_DEG)],
                        deg_hbm.at[pl.ds(base, SPAN_DEG)])


def _sc_degrees(edges):
    return pl.kernel(
        _deg_body,
        out_type=jax.ShapeDtypeStruct((2 * R * NPAD_DEG,), jnp.float32),
        mesh=_mesh(),
        scratch_types=[
            pltpu.VMEM((NROW, 128), jnp.int32),
            pltpu.VMEM((128,), jnp.float32),
            pltpu.VMEM((SPAN_DEG,), jnp.float32),
            pltpu.VMEM_SHARED((NPAD_DEG,), jnp.float32),
            pltpu.VMEM_SHARED((NPAD_DEG,), jnp.float32),
            pltpu.VMEM_SHARED((NPAD_DEG,), jnp.float32),
        ],
    )(edges)


_NZ = 16
_ZR = SPAN // _NZ


def _agg_body(h0, h1, h2, edges_hbm, agg_hbm, srcb, dstb, idxb, rows, zb,
              accum, hn):
    core = lax.axis_index("c")
    tid = lax.axis_index("s")
    hs = [h0, h1, h2]

    def _fill_z(j, carry):
        for q in range(DG // 16):
            zb[j, pl.ds(q * 16, 16)] = jnp.zeros((16,), jnp.float32)
        return carry
    lax.fori_loop(0, _ZR, _fill_z, 0)

    for r in range(R):
        for g in range(G):
            @pl.when(core == g // (G // 2))
            def _repack():
                pltpu.sync_copy(
                    hs[r].at[pl.ds(tid * SPAN, SPAN), pl.ds(g * DG, DG)],
                    hn.at[r, g, pl.ds(tid * SPAN, SPAN)])
    plsc.subcore_barrier()

    for r in range(R):
        pltpu.sync_copy(edges_hbm.at[r].at[0].at[tid], srcb)
        pltpu.sync_copy(edges_hbm.at[r].at[1].at[tid], dstb)
        for g in range(G):
            @pl.when(core == g // (G // 2))
            def _process():
                hg = hn.at[r].at[g]
                for zi in range(_NZ):
                    pltpu.sync_copy(zb, accum.at[pl.ds(tid * SPAN + zi * _ZR, _ZR)])
                plsc.subcore_barrier()

                def _gs(j, carry):
                    pltpu.sync_copy(hg.at[srcb.at[j]], rows)
                    pltpu.sync_copy(rows, accum.at[dstb.at[j]], add=True)
                    return carry
                lax.fori_loop(0, NROW, _gs, 0)
                plsc.subcore_barrier()
                pltpu.sync_copy(
                    accum.at[pl.ds(tid * SPAN, SPAN)],
                    agg_hbm.at[r].at[pl.ds(tid * SPAN, SPAN),
                                     pl.ds(g * DG, DG)])


def _sc_aggregate(h_list, edges):
    return pl.kernel(
        _agg_body,
        out_type=jax.ShapeDtypeStruct((R, PN, D), jnp.float32),
        mesh=_mesh(),
        compiler_params=pltpu.CompilerParams(use_tc_tiling_on_sc=False),
        scratch_types=[
            pltpu.VMEM((NROW, 128), jnp.int32),
            pltpu.VMEM((NROW, 128), jnp.int32),
            pltpu.VMEM((NROW, 128), jnp.int32),
            pltpu.VMEM((128, DG), jnp.float32),
            pltpu.VMEM((_ZR, DG), jnp.float32),
            pltpu.VMEM_SHARED((PN, DG), jnp.float32),
            pltpu.HBM((R, G, PN, DG), jnp.float32),
        ],
    )(h_list[0], h_list[1], h_list[2], edges)


BNM = 256
BNF = 400


def _norm(d):
    return jnp.where(d > 0, lax.rsqrt(jnp.maximum(d, 1.0)), 0.0)


def _mm_kernel(x_ref, d_ref, w_ref, o_ref):
    nrm = _norm(d_ref[...])
    o_ref[...] = jnp.dot(x_ref[...] * nrm, w_ref[...],
                         preferred_element_type=jnp.float32)


def _tc_matmul(x, dego, w):
    return pl.pallas_call(
        _mm_kernel,
        grid=(PN // BNM,),
        in_specs=[
            pl.BlockSpec((BNM, D), lambda i: (i, 0)),
            pl.BlockSpec((BNM, 1), lambda i: (i, 0)),
            pl.BlockSpec((D, D), lambda i: (0, 0)),
        ],
        out_specs=pl.BlockSpec((BNM, D), lambda i: (i, 0)),
        out_shape=jax.ShapeDtypeStruct((PN, D), jnp.float32),
    )(x, dego, w)


def _comb_kernel(relu, a_ref, d_ref, b_ref, o_ref):
    nrm = _norm(d_ref[...])
    z = (a_ref[0] * nrm[0] + a_ref[1] * nrm[1] + a_ref[2] * nrm[2]
         + b_ref[...])
    if relu:
        z = jnp.maximum(z, 0.0)
    o_ref[...] = z


def _tc_combine(aggs, degi, bias, relu, nrows, bn):
    return pl.pallas_call(
        functools.partial(_comb_kernel, relu),
        grid=(nrows // bn,),
        in_specs=[
            pl.BlockSpec((R, bn, D), lambda i: (0, i, 0)),
            pl.BlockSpec((R, bn, 1), lambda i: (0, i, 0)),
            pl.BlockSpec((1, D), lambda i: (0, 0)),
        ],
        out_specs=pl.BlockSpec((bn, D), lambda i: (i, 0)),
        out_shape=jax.ShapeDtypeStruct((nrows, D), jnp.float32),
    )(aggs, degi, bias)


def _pack_edges(e):
    pad = jnp.full((2, EPAD - E), PADIDX, jnp.int32)
    return jnp.concatenate([e, pad], axis=1).reshape(2, NS, NROW, 128)


def kernel(x, edge_index_r0, edge_index_r1, edge_index_r2,
           W1_0, b1_0, W1_1, b1_1, W1_2, b1_2,
           W2_0, b2_0, W2_1, b2_1, W2_2, b2_2):
    edges = jnp.stack([_pack_edges(edge_index_r0),
                       _pack_edges(edge_index_r1),
                       _pack_edges(edge_index_r2)])
    xp = jnp.pad(x, ((0, PN - N), (0, 0)))

    deg = _sc_degrees(edges).reshape(2, R, NPAD_DEG)
    dego = [deg[0, r, :PN, None] for r in range(R)]
    degi = deg[1, :, :PN, None]

    B1 = (b1_0 + b1_1 + b1_2)[None, :]
    B2 = (b2_0 + b2_1 + b2_2)[None, :]

    h1 = [_tc_matmul(xp, dego[r], w) for r, w in enumerate((W1_0, W1_1, W1_2))]
    agg1 = _sc_aggregate(h1, edges)
    z = _tc_combine(agg1, degi, B1, relu=True, nrows=PN, bn=BNM)
    h2 = [_tc_matmul(z, dego[r], w) for r, w in enumerate((W2_0, W2_1, W2_2))]
    agg2 = _sc_aggregate(h2, edges)
    return _tc_combine(agg2, degi[:, :N], B2, relu=False, nrows=N, bn=BNF)


# speedup vs baseline: 3.3549x; 3.3549x over previous
import functools

import jax
import jax.numpy as jnp
from jax import lax
from jax.experimental import pallas as pl
from jax.experimental.pallas import tpu as pltpu
from jax.experimental.pallas import tpu_sc as plsc

N = 50000
D = 128
E = 200000
R = 3
G = 8
DG = D // G

NS = 16
NROW = 98
EPT = NROW * 128
EPAD = NS * EPT

PN = 50432
PADIDX = 50200
SPAN = PN // NS

NPAD_DEG = 51200
SPAN_DEG = NPAD_DEG // NS


def _mesh():
    return plsc.VectorSubcoreMesh(core_axis_name="c", subcore_axis_name="s")


def _deg_body(edges_hbm, deg_hbm, idx2d, ones_v, zbuf, dac0, dac1, dac2):
    side = lax.axis_index("c")
    tid = lax.axis_index("s")
    dacs = [dac0, dac1, dac2]

    def _fill_z(q, carry):
        zbuf[pl.ds(q * 16, 16)] = jnp.zeros((16,), jnp.float32)
        return carry
    lax.fori_loop(0, SPAN_DEG // 16, _fill_z, 0)

    def _fill_o(q, carry):
        ones_v[pl.ds(q * 16, 16)] = jnp.ones((16,), jnp.float32)
        return carry
    lax.fori_loop(0, 8, _fill_o, 0)

    for a in range(R):
        pltpu.sync_copy(zbuf, dacs[a].at[pl.ds(tid * SPAN_DEG, SPAN_DEG)])
    plsc.subcore_barrier()

    for a in range(R):
        pltpu.sync_copy(edges_hbm.at[a].at[side].at[tid], idx2d)

        def _scat(j, carry):
            pltpu.sync_copy(ones_v, dacs[a].at[idx2d.at[j]], add=True)
            return carry
        lax.fori_loop(0, NROW, _scat, 0)
    plsc.subcore_barrier()

    for a in range(R):
        base = (side * R + a) * NPAD_DEG + tid * SPAN_DEG
        pltpu.sync_copy(dacs[a].at[pl.ds(tid * SPAN_DEG, SPAN_DEG)],
                        deg_hbm.at[pl.ds(base, SPAN_DEG)])


def _sc_degrees(edges):
    return pl.kernel(
        _deg_body,
        out_type=jax.ShapeDtypeStruct((2 * R * NPAD_DEG,), jnp.float32),
        mesh=_mesh(),
        scratch_types=[
            pltpu.VMEM((NROW, 128), jnp.int32),
            pltpu.VMEM((128,), jnp.float32),
            pltpu.VMEM((SPAN_DEG,), jnp.float32),
            pltpu.VMEM_SHARED((NPAD_DEG,), jnp.float32),
            pltpu.VMEM_SHARED((NPAD_DEG,), jnp.float32),
            pltpu.VMEM_SHARED((NPAD_DEG,), jnp.float32),
        ],
    )(edges)


_NZ = 16
_ZR = SPAN // _NZ


def _agg_body(h0, h1, h2, edges_hbm, agg_hbm, srcb, dstb, rowsa, rowsb, zb,
              accum, gsa, gsb):
    core = lax.axis_index("c")
    tid = lax.axis_index("s")
    hs = [h0, h1, h2]

    def _fill_z(j, carry):
        for q in range(DG // 16):
            zb[j, pl.ds(q * 16, 16)] = jnp.zeros((16,), jnp.float32)
        return carry
    lax.fori_loop(0, _ZR, _fill_z, 0)

    for r in range(R):
        pltpu.sync_copy(edges_hbm.at[r].at[0].at[tid], srcb)
        pltpu.sync_copy(edges_hbm.at[r].at[1].at[tid], dstb)
        for g in range(G):
            @pl.when(core == g // (G // 2))
            def _process():
                hg = hs[r].at[g]
                for zi in range(_NZ):
                    pltpu.sync_copy(zb, accum.at[pl.ds(tid * SPAN + zi * _ZR, _ZR)])
                plsc.subcore_barrier()

                pltpu.async_copy(hg.at[srcb.at[0]], rowsa, gsa)

                def _gs(i, carry):
                    j0 = 2 * i
                    pltpu.async_copy(hg.at[srcb.at[j0 + 1]], rowsb, gsb)
                    pltpu.make_async_copy(hg.at[srcb.at[j0]], rowsa, gsa).wait()
                    pltpu.sync_copy(rowsa, accum.at[dstb.at[j0]], add=True)

                    @pl.when(i < NROW // 2 - 1)
                    def _pref():
                        pltpu.async_copy(hg.at[srcb.at[j0 + 2]], rowsa, gsa)
                    pltpu.make_async_copy(hg.at[srcb.at[j0 + 1]], rowsb, gsb).wait()
                    pltpu.sync_copy(rowsb, accum.at[dstb.at[j0 + 1]], add=True)
                    return carry
                lax.fori_loop(0, NROW // 2, _gs, 0)
                plsc.subcore_barrier()
                pltpu.sync_copy(
                    accum.at[pl.ds(tid * SPAN, SPAN)],
                    agg_hbm.at[r].at[pl.ds(tid * SPAN, SPAN),
                                     pl.ds(g * DG, DG)])


def _sc_aggregate(h_list, edges):
    return pl.kernel(
        _agg_body,
        out_type=jax.ShapeDtypeStruct((R, PN, D), jnp.float32),
        mesh=_mesh(),
        compiler_params=pltpu.CompilerParams(use_tc_tiling_on_sc=False),
        scratch_types=[
            pltpu.VMEM((NROW, 128), jnp.int32),
            pltpu.VMEM((NROW, 128), jnp.int32),
            pltpu.VMEM((128, DG), jnp.float32),
            pltpu.VMEM((128, DG), jnp.float32),
            pltpu.VMEM((_ZR, DG), jnp.float32),
            pltpu.VMEM_SHARED((PN, DG), jnp.float32),
            pltpu.SemaphoreType.DMA,
            pltpu.SemaphoreType.DMA,
        ],
    )(h_list[0], h_list[1], h_list[2], edges)


BNM = 256
BNF = 400


def _norm(d):
    return jnp.where(d > 0, lax.rsqrt(jnp.maximum(d, 1.0)), 0.0)


def _mm_kernel(x_ref, d_ref, w_ref, o_ref):
    nrm = _norm(d_ref[...])
    h = jnp.dot(x_ref[...] * nrm, w_ref[...],
                preferred_element_type=jnp.float32)
    for g in range(G):
        o_ref[g] = h[:, g * DG:(g + 1) * DG]


def _tc_matmul(x, dego, w):
    return pl.pallas_call(
        _mm_kernel,
        grid=(PN // BNM,),
        in_specs=[
            pl.BlockSpec((BNM, D), lambda i: (i, 0)),
            pl.BlockSpec((BNM, 1), lambda i: (i, 0)),
            pl.BlockSpec((D, D), lambda i: (0, 0)),
        ],
        out_specs=pl.BlockSpec((G, BNM, DG), lambda i: (0, i, 0)),
        out_shape=jax.ShapeDtypeStruct((G, PN, DG), jnp.float32),
    )(x, dego, w)


def _comb_kernel(relu, a_ref, d_ref, b_ref, o_ref):
    nrm = _norm(d_ref[...])
    z = (a_ref[0] * nrm[0] + a_ref[1] * nrm[1] + a_ref[2] * nrm[2]
         + b_ref[...])
    if relu:
        z = jnp.maximum(z, 0.0)
    o_ref[...] = z


def _tc_combine(aggs, degi, bias, relu, nrows, bn):
    return pl.pallas_call(
        functools.partial(_comb_kernel, relu),
        grid=(nrows // bn,),
        in_specs=[
            pl.BlockSpec((R, bn, D), lambda i: (0, i, 0)),
            pl.BlockSpec((R, bn, 1), lambda i: (0, i, 0)),
            pl.BlockSpec((1, D), lambda i: (0, 0)),
        ],
        out_specs=pl.BlockSpec((bn, D), lambda i: (i, 0)),
        out_shape=jax.ShapeDtypeStruct((nrows, D), jnp.float32),
    )(aggs, degi, bias)


def _pack_edges(e):
    pad = jnp.full((2, EPAD - E), PADIDX, jnp.int32)
    return jnp.concatenate([e, pad], axis=1).reshape(2, NS, NROW, 128)


def kernel(x, edge_index_r0, edge_index_r1, edge_index_r2,
           W1_0, b1_0, W1_1, b1_1, W1_2, b1_2,
           W2_0, b2_0, W2_1, b2_1, W2_2, b2_2):
    edges = jnp.stack([_pack_edges(edge_index_r0),
                       _pack_edges(edge_index_r1),
                       _pack_edges(edge_index_r2)])
    xp = jnp.pad(x, ((0, PN - N), (0, 0)))

    deg = _sc_degrees(edges).reshape(2, R, NPAD_DEG)
    dego = [deg[0, r, :PN, None] for r in range(R)]
    degi = deg[1, :, :PN, None]

    B1 = (b1_0 + b1_1 + b1_2)[None, :]
    B2 = (b2_0 + b2_1 + b2_2)[None, :]

    h1 = [_tc_matmul(xp, dego[r], w) for r, w in enumerate((W1_0, W1_1, W1_2))]
    agg1 = _sc_aggregate(h1, edges)
    z = _tc_combine(agg1, degi, B1, relu=True, nrows=PN, bn=BNM)
    h2 = [_tc_matmul(z, dego[r], w) for r, w in enumerate((W2_0, W2_1, W2_2))]
    agg2 = _sc_aggregate(h2, edges)
    return _tc_combine(agg2, degi[:, :N], B2, relu=False, nrows=N, bn=BNF)


# speedup vs baseline: 4.4183x; 1.3170x over previous
import functools

import jax
import jax.numpy as jnp
from jax import lax
from jax.experimental import pallas as pl
from jax.experimental.pallas import tpu as pltpu
from jax.experimental.pallas import tpu_sc as plsc

N = 50000
D = 128
E = 200000
R = 3
G = 8
DG = D // G

NS = 16
NROW = 98
EPT = NROW * 128
EPAD = NS * EPT

PN = 50432
PADIDX = 50200
SPAN = PN // NS

NPAD_DEG = 51200
SPAN_DEG = NPAD_DEG // NS


def _mesh():
    return plsc.VectorSubcoreMesh(core_axis_name="c", subcore_axis_name="s")


def _deg_body(edges_hbm, deg_hbm, idx2d, ones_v, zbuf, dac0, dac1, dac2):
    side = lax.axis_index("c")
    tid = lax.axis_index("s")
    dacs = [dac0, dac1, dac2]

    def _fill_z(q, carry):
        zbuf[pl.ds(q * 16, 16)] = jnp.zeros((16,), jnp.float32)
        return carry
    lax.fori_loop(0, SPAN_DEG // 16, _fill_z, 0)

    def _fill_o(q, carry):
        ones_v[pl.ds(q * 16, 16)] = jnp.ones((16,), jnp.float32)
        return carry
    lax.fori_loop(0, 8, _fill_o, 0)

    for a in range(R):
        pltpu.sync_copy(zbuf, dacs[a].at[pl.ds(tid * SPAN_DEG, SPAN_DEG)])
    plsc.subcore_barrier()

    for a in range(R):
        pltpu.sync_copy(edges_hbm.at[a].at[side].at[tid], idx2d)

        def _scat(j, carry):
            pltpu.sync_copy(ones_v, dacs[a].at[idx2d.at[j]], add=True)
            return carry
        lax.fori_loop(0, NROW, _scat, 0)
    plsc.subcore_barrier()

    for a in range(R):
        base = (side * R + a) * NPAD_DEG + tid * SPAN_DEG
        pltpu.sync_copy(dacs[a].at[pl.ds(tid * SPAN_DEG, SPAN_DEG)],
                        deg_hbm.at[pl.ds(base, SPAN_DEG)])


def _sc_degrees(edges):
    return pl.kernel(
        _deg_body,
        out_type=jax.ShapeDtypeStruct((2 * R * NPAD_DEG,), jnp.float32),
        mesh=_mesh(),
        scratch_types=[
            pltpu.VMEM((NROW, 128), jnp.int32),
            pltpu.VMEM((128,), jnp.float32),
            pltpu.VMEM((SPAN_DEG,), jnp.float32),
            pltpu.VMEM_SHARED((NPAD_DEG,), jnp.float32),
            pltpu.VMEM_SHARED((NPAD_DEG,), jnp.float32),
            pltpu.VMEM_SHARED((NPAD_DEG,), jnp.float32),
        ],
    )(edges)


_NZ = 16
_ZR = SPAN // _NZ


def _agg_body(h, edges_hbm, agg_hbm, srcb, dstb, rowsa, rowsb, zb,
              accum, gsa, gsb):
    core = lax.axis_index("c")
    tid = lax.axis_index("s")

    def _fill_z(j, carry):
        for q in range(DG // 16):
            zb[j, pl.ds(q * 16, 16)] = jnp.zeros((16,), jnp.float32)
        return carry
    lax.fori_loop(0, _ZR, _fill_z, 0)

    pltpu.sync_copy(edges_hbm.at[0].at[tid], srcb)
    pltpu.sync_copy(edges_hbm.at[1].at[tid], dstb)
    for g in range(G):
        @pl.when(core == g // (G // 2))
        def _process():
            hg = h.at[g]
            for zi in range(_NZ):
                pltpu.sync_copy(zb, accum.at[pl.ds(tid * SPAN + zi * _ZR, _ZR)])
            plsc.subcore_barrier()

            pltpu.async_copy(hg.at[srcb.at[0]], rowsa, gsa)

            def _gs(i, carry):
                j0 = 2 * i
                pltpu.async_copy(hg.at[srcb.at[j0 + 1]], rowsb, gsb)
                pltpu.make_async_copy(hg.at[srcb.at[j0]], rowsa, gsa).wait()
                pltpu.sync_copy(rowsa, accum.at[dstb.at[j0]], add=True)

                @pl.when(i < NROW // 2 - 1)
                def _pref():
                    pltpu.async_copy(hg.at[srcb.at[j0 + 2]], rowsa, gsa)
                pltpu.make_async_copy(hg.at[srcb.at[j0 + 1]], rowsb, gsb).wait()
                pltpu.sync_copy(rowsb, accum.at[dstb.at[j0 + 1]], add=True)
                return carry
            lax.fori_loop(0, NROW // 2, _gs, 0)
            plsc.subcore_barrier()
            pltpu.sync_copy(
                accum.at[pl.ds(tid * SPAN, SPAN)],
                agg_hbm.at[pl.ds(tid * SPAN, SPAN), pl.ds(g * DG, DG)])


def _sc_aggregate(h, edges):
    return pl.kernel(
        _agg_body,
        out_type=jax.ShapeDtypeStruct((PN, D), jnp.float32),
        mesh=_mesh(),
        compiler_params=pltpu.CompilerParams(use_tc_tiling_on_sc=False),
        scratch_types=[
            pltpu.VMEM((NROW, 128), jnp.int32),
            pltpu.VMEM((NROW, 128), jnp.int32),
            pltpu.VMEM((128, DG), jnp.float32),
            pltpu.VMEM((128, DG), jnp.float32),
            pltpu.VMEM((_ZR, DG), jnp.float32),
            pltpu.VMEM_SHARED((PN, DG), jnp.float32),
            pltpu.SemaphoreType.DMA,
            pltpu.SemaphoreType.DMA,
        ],
    )(h, edges)


BNM = 256
BNF = 400


def _norm(d):
    return jnp.where(d > 0, lax.rsqrt(jnp.maximum(d, 1.0)), 0.0)


def _mm_kernel(x_ref, d_ref, w_ref, o_ref):
    nrm = _norm(d_ref[...])
    h = jnp.dot(x_ref[...] * nrm, w_ref[...],
                preferred_element_type=jnp.float32)
    for g in range(G):
        o_ref[g] = h[:, g * DG:(g + 1) * DG]


def _tc_matmul(x, dego, w):
    return pl.pallas_call(
        _mm_kernel,
        grid=(PN // BNM,),
        in_specs=[
            pl.BlockSpec((BNM, D), lambda i: (i, 0)),
            pl.BlockSpec((BNM, 1), lambda i: (i, 0)),
            pl.BlockSpec((D, D), lambda i: (0, 0)),
        ],
        out_specs=pl.BlockSpec((G, BNM, DG), lambda i: (0, i, 0)),
        out_shape=jax.ShapeDtypeStruct((G, PN, DG), jnp.float32),
    )(x, dego, w)


def _comb_kernel(relu, a0_ref, a1_ref, a2_ref, d_ref, b_ref, o_ref):
    nrm = _norm(d_ref[...])
    z = (a0_ref[...] * nrm[0] + a1_ref[...] * nrm[1] + a2_ref[...] * nrm[2]
         + b_ref[...])
    if relu:
        z = jnp.maximum(z, 0.0)
    o_ref[...] = z


def _tc_combine(aggs, degi, bias, relu, nrows, bn):
    return pl.pallas_call(
        functools.partial(_comb_kernel, relu),
        grid=(nrows // bn,),
        in_specs=[
            pl.BlockSpec((bn, D), lambda i: (i, 0)),
            pl.BlockSpec((bn, D), lambda i: (i, 0)),
            pl.BlockSpec((bn, D), lambda i: (i, 0)),
            pl.BlockSpec((R, bn, 1), lambda i: (0, i, 0)),
            pl.BlockSpec((1, D), lambda i: (0, 0)),
        ],
        out_specs=pl.BlockSpec((bn, D), lambda i: (i, 0)),
        out_shape=jax.ShapeDtypeStruct((nrows, D), jnp.float32),
    )(aggs[0], aggs[1], aggs[2], degi, bias)


def _pack_edges(e):
    pad = jnp.full((2, EPAD - E), PADIDX, jnp.int32)
    return jnp.concatenate([e, pad], axis=1).reshape(2, NS, NROW, 128)


def kernel(x, edge_index_r0, edge_index_r1, edge_index_r2,
           W1_0, b1_0, W1_1, b1_1, W1_2, b1_2,
           W2_0, b2_0, W2_1, b2_1, W2_2, b2_2):
    edges_r = [_pack_edges(edge_index_r0), _pack_edges(edge_index_r1),
               _pack_edges(edge_index_r2)]
    edges = jnp.stack(edges_r)
    xp = jnp.pad(x, ((0, PN - N), (0, 0)))

    deg = _sc_degrees(edges).reshape(2, R, NPAD_DEG)
    dego = [deg[0, r, :PN, None] for r in range(R)]
    degi = deg[1, :, :PN, None]

    B1 = (b1_0 + b1_1 + b1_2)[None, :]
    B2 = (b2_0 + b2_1 + b2_2)[None, :]

    h1 = [_tc_matmul(xp, dego[r], w) for r, w in enumerate((W1_0, W1_1, W1_2))]
    agg1 = [_sc_aggregate(h1[r], edges_r[r]) for r in range(R)]
    z = _tc_combine(agg1, degi, B1, relu=True, nrows=PN, bn=BNM)
    h2 = [_tc_matmul(z, dego[r], w) for r, w in enumerate((W2_0, W2_1, W2_2))]
    agg2 = [_sc_aggregate(h2[r], edges_r[r]) for r in range(R)]
    return _tc_combine(agg2, degi[:, :N], B2, relu=False, nrows=N, bn=BNF)


# speedup vs baseline: 4.4926x; 1.0168x over previous
import functools

import jax
import jax.numpy as jnp
from jax import lax
from jax.experimental import pallas as pl
from jax.experimental.pallas import tpu as pltpu
from jax.experimental.pallas import tpu_sc as plsc

N = 50000
D = 128
E = 200000
R = 3
G = 8
DG = D // G

NS = 16
NROW = 98
EPT = NROW * 128
EPAD = NS * EPT

PN = 50432
PADIDX = 50200
SPAN = PN // NS

NPAD_DEG = 51200
SPAN_DEG = NPAD_DEG // NS


def _mesh():
    return plsc.VectorSubcoreMesh(core_axis_name="c", subcore_axis_name="s")


def _deg_body(edges_hbm, deg_hbm, idx2d, ones_v, zbuf, dac0, dac1, dac2):
    side = lax.axis_index("c")
    tid = lax.axis_index("s")
    dacs = [dac0, dac1, dac2]

    def _fill_z(q, carry):
        zbuf[pl.ds(q * 16, 16)] = jnp.zeros((16,), jnp.float32)
        return carry
    lax.fori_loop(0, SPAN_DEG // 16, _fill_z, 0)

    def _fill_o(q, carry):
        ones_v[pl.ds(q * 16, 16)] = jnp.ones((16,), jnp.float32)
        return carry
    lax.fori_loop(0, 8, _fill_o, 0)

    for a in range(R):
        pltpu.sync_copy(zbuf, dacs[a].at[pl.ds(tid * SPAN_DEG, SPAN_DEG)])
    plsc.subcore_barrier()

    for a in range(R):
        pltpu.sync_copy(edges_hbm.at[a].at[side].at[tid], idx2d)

        def _scat(j, carry):
            pltpu.sync_copy(ones_v, dacs[a].at[idx2d.at[j]], add=True)
            return carry
        lax.fori_loop(0, NROW, _scat, 0)
    plsc.subcore_barrier()

    for a in range(R):
        base = (side * R + a) * NPAD_DEG + tid * SPAN_DEG
        pltpu.sync_copy(dacs[a].at[pl.ds(tid * SPAN_DEG, SPAN_DEG)],
                        deg_hbm.at[pl.ds(base, SPAN_DEG)])


def _sc_degrees(edges):
    return pl.kernel(
        _deg_body,
        out_type=jax.ShapeDtypeStruct((2 * R * NPAD_DEG,), jnp.float32),
        mesh=_mesh(),
        scratch_types=[
            pltpu.VMEM((NROW, 128), jnp.int32),
            pltpu.VMEM((128,), jnp.float32),
            pltpu.VMEM((SPAN_DEG,), jnp.float32),
            pltpu.VMEM_SHARED((NPAD_DEG,), jnp.float32),
            pltpu.VMEM_SHARED((NPAD_DEG,), jnp.float32),
            pltpu.VMEM_SHARED((NPAD_DEG,), jnp.float32),
        ],
    )(edges)


_NZ = 16
_ZR = SPAN // _NZ


def _agg_body(h, edges_hbm, agg_hbm, srcb, dstb, r0, r1, r2, r3, zb,
              accum, g0, g1, g2, g3, s0, s1, s2, s3):
    core = lax.axis_index("c")
    tid = lax.axis_index("s")
    bufs = [r0, r1, r2, r3]
    gsem = [g0, g1, g2, g3]
    ssem = [s0, s1, s2, s3]

    def _fill_z(j, carry):
        for q in range(DG // 16):
            zb[j, pl.ds(q * 16, 16)] = jnp.zeros((16,), jnp.float32)
        return carry
    lax.fori_loop(0, _ZR, _fill_z, 0)

    pltpu.sync_copy(edges_hbm.at[0].at[tid], srcb)
    pltpu.sync_copy(edges_hbm.at[1].at[tid], dstb)
    for g in range(G):
        @pl.when(core == g // (G // 2))
        def _process():
            hg = h.at[g]
            for zi in range(_NZ):
                pltpu.sync_copy(zb, accum.at[pl.ds(tid * SPAN + zi * _ZR, _ZR)])
            plsc.subcore_barrier()

            pltpu.async_copy(hg.at[srcb.at[0]], bufs[0], gsem[0])
            pltpu.async_copy(hg.at[srcb.at[1]], bufs[1], gsem[1])

            def _quad(q, carry):
                for k in range(4):
                    j = 4 * q + k
                    kn = (k + 2) % 4
                    pltpu.make_async_copy(hg.at[srcb.at[j]],
                                          bufs[k], gsem[k]).wait()
                    pltpu.async_copy(bufs[k], accum.at[dstb.at[j]],
                                     ssem[k], add=True)
                    if k < 2:
                        @pl.when(q > 0)
                        def _w():
                            pltpu.make_async_copy(
                                bufs[kn], accum.at[dstb.at[j - 2]],
                                ssem[kn]).wait()
                    else:
                        pltpu.make_async_copy(
                            bufs[kn], accum.at[dstb.at[j - 2]],
                            ssem[kn]).wait()
                    pltpu.async_copy(hg.at[srcb.at[j + 2]], bufs[kn], gsem[kn])
                return carry
            lax.fori_loop(0, (NROW - 2) // 4, _quad, 0)

            for (j, k) in ((NROW - 2, 0), (NROW - 1, 1)):
                pltpu.make_async_copy(hg.at[srcb.at[j]],
                                      bufs[k], gsem[k]).wait()
                pltpu.sync_copy(bufs[k], accum.at[dstb.at[j]], add=True)
            for k in (2, 3):
                pltpu.make_async_copy(bufs[k], accum.at[dstb.at[NROW - 4 + k - 2]],
                                      ssem[k]).wait()
            plsc.subcore_barrier()
            pltpu.sync_copy(
                accum.at[pl.ds(tid * SPAN, SPAN)],
                agg_hbm.at[pl.ds(tid * SPAN, SPAN), pl.ds(g * DG, DG)])


def _sc_aggregate(h, edges):
    return pl.kernel(
        _agg_body,
        out_type=jax.ShapeDtypeStruct((PN, D), jnp.float32),
        mesh=_mesh(),
        compiler_params=pltpu.CompilerParams(use_tc_tiling_on_sc=False),
        scratch_types=[
            pltpu.VMEM((NROW, 128), jnp.int32),
            pltpu.VMEM((NROW, 128), jnp.int32),
            pltpu.VMEM((128, DG), jnp.float32),
            pltpu.VMEM((128, DG), jnp.float32),
            pltpu.VMEM((128, DG), jnp.float32),
            pltpu.VMEM((128, DG), jnp.float32),
            pltpu.VMEM((_ZR, DG), jnp.float32),
            pltpu.VMEM_SHARED((PN, DG), jnp.float32),
            pltpu.SemaphoreType.DMA, pltpu.SemaphoreType.DMA,
            pltpu.SemaphoreType.DMA, pltpu.SemaphoreType.DMA,
            pltpu.SemaphoreType.DMA, pltpu.SemaphoreType.DMA,
            pltpu.SemaphoreType.DMA, pltpu.SemaphoreType.DMA,
        ],
    )(h, edges)


BNM = 256
BNF = 400


def _norm(d):
    return jnp.where(d > 0, lax.rsqrt(jnp.maximum(d, 1.0)), 0.0)


def _mm_kernel(x_ref, d_ref, w_ref, o_ref):
    nrm = _norm(d_ref[...])
    h = jnp.dot(x_ref[...] * nrm, w_ref[...],
                preferred_element_type=jnp.float32)
    for g in range(G):
        o_ref[g] = h[:, g * DG:(g + 1) * DG]


def _tc_matmul(x, dego, w):
    return pl.pallas_call(
        _mm_kernel,
        grid=(PN // BNM,),
        in_specs=[
            pl.BlockSpec((BNM, D), lambda i: (i, 0)),
            pl.BlockSpec((BNM, 1), lambda i: (i, 0)),
            pl.BlockSpec((D, D), lambda i: (0, 0)),
        ],
        out_specs=pl.BlockSpec((G, BNM, DG), lambda i: (0, i, 0)),
        out_shape=jax.ShapeDtypeStruct((G, PN, DG), jnp.float32),
    )(x, dego, w)


def _comb_kernel(relu, a0_ref, a1_ref, a2_ref, d_ref, b_ref, o_ref):
    nrm = _norm(d_ref[...])
    z = (a0_ref[...] * nrm[0] + a1_ref[...] * nrm[1] + a2_ref[...] * nrm[2]
         + b_ref[...])
    if relu:
        z = jnp.maximum(z, 0.0)
    o_ref[...] = z


def _tc_combine(aggs, degi, bias, relu, nrows, bn):
    return pl.pallas_call(
        functools.partial(_comb_kernel, relu),
        grid=(nrows // bn,),
        in_specs=[
            pl.BlockSpec((bn, D), lambda i: (i, 0)),
            pl.BlockSpec((bn, D), lambda i: (i, 0)),
            pl.BlockSpec((bn, D), lambda i: (i, 0)),
            pl.BlockSpec((R, bn, 1), lambda i: (0, i, 0)),
            pl.BlockSpec((1, D), lambda i: (0, 0)),
        ],
        out_specs=pl.BlockSpec((bn, D), lambda i: (i, 0)),
        out_shape=jax.ShapeDtypeStruct((nrows, D), jnp.float32),
    )(aggs[0], aggs[1], aggs[2], degi, bias)


def _pack_edges(e):
    pad = jnp.full((2, EPAD - E), PADIDX, jnp.int32)
    return jnp.concatenate([e, pad], axis=1).reshape(2, NS, NROW, 128)


def kernel(x, edge_index_r0, edge_index_r1, edge_index_r2,
           W1_0, b1_0, W1_1, b1_1, W1_2, b1_2,
           W2_0, b2_0, W2_1, b2_1, W2_2, b2_2):
    edges_r = [_pack_edges(edge_index_r0), _pack_edges(edge_index_r1),
               _pack_edges(edge_index_r2)]
    edges = jnp.stack(edges_r)
    xp = jnp.pad(x, ((0, PN - N), (0, 0)))

    deg = _sc_degrees(edges).reshape(2, R, NPAD_DEG)
    dego = [deg[0, r, :PN, None] for r in range(R)]
    degi = deg[1, :, :PN, None]

    B1 = (b1_0 + b1_1 + b1_2)[None, :]
    B2 = (b2_0 + b2_1 + b2_2)[None, :]

    h1 = [_tc_matmul(xp, dego[r], w) for r, w in enumerate((W1_0, W1_1, W1_2))]
    agg1 = [_sc_aggregate(h1[r], edges_r[r]) for r in range(R)]
    z = _tc_combine(agg1, degi, B1, relu=True, nrows=PN, bn=BNM)
    h2 = [_tc_matmul(z, dego[r], w) for r, w in enumerate((W2_0, W2_1, W2_2))]
    agg2 = [_sc_aggregate(h2[r], edges_r[r]) for r in range(R)]
    return _tc_combine(agg2, degi[:, :N], B2, relu=False, nrows=N, bn=BNF)
